# Initial kernel scaffold; baseline (speedup 1.0000x reference)
import jax
import jax.numpy as jnp
from jax.experimental import pallas as pl


def kernel(x, A_neg, A_pos, W1, b1, W2, b2, W3, b3, Wg1, bg1, Wg2, bg2, Wg3, bg3, Wg4, bg4, Wg5, bg5, Wg6, bg6):
    raise NotImplementedError("write your pallas kernel here")


# speedup vs baseline: 967.9279x; 967.9279x over previous
import jax
import jax.numpy as jnp
from jax.experimental import pallas as pl

_HIGH = jax.lax.Precision.HIGHEST


def _matmul(a, b):
    return jax.lax.dot_general(a, b, (((1,), (0,)), ((), ())),
                               precision=_HIGH,
                               preferred_element_type=jnp.float32)


def _matmul_ta(a, b):
    return jax.lax.dot_general(a, b, (((0,), (0,)), ((), ())),
                               precision=_HIGH,
                               preferred_element_type=jnp.float32)


def _body(x_ref, A_ref, W1_ref, b1_ref, W2_ref, b2_ref, W3_ref, b3_ref,
          Wg1_ref, bg1_ref, Wg2_ref, bg2_ref, Wg3_ref, bg3_ref,
          Wg4_ref, bg4_ref, Wg5_ref, bg5_ref, Wg6_ref, bg6_ref, out_ref):
    n = A_ref.shape[0]
    M = (A_ref[...] != 0).astype(jnp.float32)
    ones = jnp.ones((n, 1), jnp.float32)
    deg = _matmul_ta(M, ones) + 1.0
    dinv = jax.lax.rsqrt(deg)
    dinv2 = dinv * dinv

    def gcn(h, W_ref, b_ref):
        hw = _matmul(h, W_ref[...])
        t = _matmul_ta(M, hw * dinv)
        return t * dinv + hw * dinv2 + b_ref[...]

    x = x_ref[...]
    x1l = _matmul(x, W1_ref[...]) + b1_ref[...]
    x1 = x1l + jax.nn.relu(gcn(x1l, Wg1_ref, bg1_ref))
    x2l = _matmul(x1, W2_ref[...]) + b2_ref[...]
    x2 = x2l + jax.nn.relu(gcn(x2l, Wg2_ref, bg2_ref))
    x3l = _matmul(x2, W3_ref[...]) + b3_ref[...]
    x3 = x3l + 0.5 * jax.nn.relu(gcn(x3l, Wg3_ref, bg3_ref))
    x4 = x3 + 0.5 * jax.nn.relu(gcn(x3, Wg4_ref, bg4_ref))
    x5 = x4 + 0.25 * jax.nn.relu(gcn(x4, Wg5_ref, bg5_ref))
    out_ref[...] = x5 + 0.25 * gcn(x5, Wg6_ref, bg6_ref)


def kernel(x, A_neg, A_pos, W1, b1, W2, b2, W3, b3, Wg1, bg1, Wg2, bg2,
           Wg3, bg3, Wg4, bg4, Wg5, bg5, Wg6, bg6):
    del A_pos
    n, dout = x.shape[0], Wg3.shape[0]

    d2 = W2.shape[1]
    pad = dout - d2
    W2p = jnp.pad(W2, ((0, 0), (0, pad)))
    b2p = jnp.pad(b2, (0, pad))
    Wg2p = jnp.pad(Wg2, ((0, pad), (0, pad)))
    bg2p = jnp.pad(bg2, (0, pad))
    W3p = jnp.pad(W3, ((0, pad), (0, 0)))

    row = lambda v: v.reshape(1, -1)
    return pl.pallas_call(
        _body,
        out_shape=jax.ShapeDtypeStruct((n, dout), jnp.float32),
    )(x, A_neg, W1, row(b1), W2p, row(b2p), W3p, row(b3),
      Wg1, row(bg1), Wg2p, row(bg2p), Wg3, row(bg3),
      Wg4, row(bg4), Wg5, row(bg5), Wg6, row(bg6))


# speedup vs baseline: 1978.8447x; 2.0444x over previous
import jax
import jax.numpy as jnp
from jax.experimental import pallas as pl


def _matmul(a, b):
    return jax.lax.dot_general(a, b, (((1,), (0,)), ((), ())),
                               precision=jax.lax.Precision.HIGHEST,
                               preferred_element_type=jnp.float32)


def _matmul_ta_bf16(a, b):
    return jax.lax.dot_general(a, b, (((0,), (0,)), ((), ())),
                               preferred_element_type=jnp.float32)


def _body(x_ref, M_ref, W1_ref, b1_ref, W2_ref, b2_ref, W3_ref, b3_ref,
          Wg1_ref, bg1_ref, Wg2_ref, bg2_ref, Wg3_ref, bg3_ref,
          Wg4_ref, bg4_ref, Wg5_ref, bg5_ref, Wg6_ref, bg6_ref, out_ref):
    n = M_ref.shape[0]
    M = M_ref[...]
    ones = jnp.ones((n, 1), jnp.bfloat16)
    deg = _matmul_ta_bf16(M, ones) + 1.0
    dinv = jax.lax.rsqrt(deg)
    dinv2 = dinv * dinv

    def gcn(h, W_ref, b_ref):
        hw = _matmul(h, W_ref[...])
        t = _matmul_ta_bf16(M, (hw * dinv).astype(jnp.bfloat16))
        return t * dinv + hw * dinv2 + b_ref[...]

    x = x_ref[...]
    x1l = _matmul(x, W1_ref[...]) + b1_ref[...]
    x1 = x1l + jax.nn.relu(gcn(x1l, Wg1_ref, bg1_ref))
    x2l = _matmul(x1, W2_ref[...]) + b2_ref[...]
    x2 = x2l + jax.nn.relu(gcn(x2l, Wg2_ref, bg2_ref))
    x3l = _matmul(x2, W3_ref[...]) + b3_ref[...]
    x3 = x3l + 0.5 * jax.nn.relu(gcn(x3l, Wg3_ref, bg3_ref))
    x4 = x3 + 0.5 * jax.nn.relu(gcn(x3, Wg4_ref, bg4_ref))
    x5 = x4 + 0.25 * jax.nn.relu(gcn(x4, Wg5_ref, bg5_ref))
    out_ref[...] = x5 + 0.25 * gcn(x5, Wg6_ref, bg6_ref)


def kernel(x, A_neg, A_pos, W1, b1, W2, b2, W3, b3, Wg1, bg1, Wg2, bg2,
           Wg3, bg3, Wg4, bg4, Wg5, bg5, Wg6, bg6):
    del A_pos
    n, dout = x.shape[0], Wg3.shape[0]

    Mbf = (A_neg != 0).astype(jnp.bfloat16)

    d2 = W2.shape[1]
    pad = dout - d2
    W2p = jnp.pad(W2, ((0, 0), (0, pad)))
    b2p = jnp.pad(b2, (0, pad))
    Wg2p = jnp.pad(Wg2, ((0, pad), (0, pad)))
    bg2p = jnp.pad(bg2, (0, pad))
    W3p = jnp.pad(W3, ((0, pad), (0, 0)))

    row = lambda v: v.reshape(1, -1)
    return pl.pallas_call(
        _body,
        out_shape=jax.ShapeDtypeStruct((n, dout), jnp.float32),
    )(x, Mbf, W1, row(b1), W2p, row(b2p), W3p, row(b3),
      Wg1, row(bg1), Wg2p, row(bg2p), Wg3, row(bg3),
      Wg4, row(bg4), Wg5, row(bg5), Wg6, row(bg6))
